# Initial kernel scaffold; baseline (speedup 1.0000x reference)
import jax
import jax.numpy as jnp
from jax.experimental import pallas as pl


def kernel(node_feats, edge_feats, edge_index, W_enc, b_enc, Ws, bs, W_dec, b_dec):
    raise NotImplementedError("write your pallas kernel here")


# speedup vs baseline: 4.2690x; 4.2690x over previous
import functools

import jax
import jax.numpy as jnp
from jax import lax
from jax.experimental import pallas as pl
from jax.experimental.pallas import tpu as pltpu
from jax.experimental.pallas import tpu_sc as plsc

N = 10000
E = 320000
H = 128
L = 10
NC = 2
NS = 16
NW = NC * NS
EPW = E // NW
K = 128
NFULL = EPW // K
TAIL = EPW - NFULL * K
ZROWS = 624

_mesh = lambda: plsc.VectorSubcoreMesh(core_axis_name="c", subcore_axis_name="s")

FP32 = jnp.float32
I32 = jnp.int32


def _seg_sum(g, src, dst):
    @functools.partial(
        pl.kernel,
        out_type=jax.ShapeDtypeStruct((NC, N, H), FP32),
        mesh=_mesh(),
        compiler_params=pltpu.CompilerParams(use_tc_tiling_on_sc=False),
        scratch_types=[
            pltpu.VMEM((K,), I32),
            pltpu.VMEM((K,), I32),
            pltpu.VMEM((K, H), FP32),
            pltpu.VMEM((TAIL,), I32),
            pltpu.VMEM((TAIL,), I32),
            pltpu.VMEM((TAIL, H), FP32),
            pltpu.VMEM((8, H), FP32),
            pltpu.VMEM_SHARED((N, H), FP32),
            pltpu.SemaphoreType.DMA,
        ],
    )
    def k(g_hbm, src_hbm, dst_hbm, out_hbm,
          sidx, didx, rows, tsidx, tdidx, trows, zbuf, accum, sem):
        c = lax.axis_index("c")
        s = lax.axis_index("s")
        wid = s * NC + c

        zero16 = jnp.zeros((16,), FP32)
        for r in range(8):
            for q in range(H // 16):
                zbuf[r, pl.ds(q * 16, 16)] = zero16

        zbase = s * ZROWS

        def zloop(i, _):
            pltpu.sync_copy(zbuf, accum.at[pl.ds(zbase + i * 8, 8)])
            return 0

        lax.fori_loop(0, ZROWS // 8, zloop, 0)

        @pl.when(s == NS - 1)
        def _():
            pltpu.sync_copy(zbuf, accum.at[pl.ds(16 * ZROWS, 8)])
            pltpu.sync_copy(zbuf, accum.at[pl.ds(16 * ZROWS + 8, 8)])

        plsc.subcore_barrier()

        ebase = wid * EPW

        def chunk(i, _):
            off = ebase + i * K
            pltpu.sync_copy(src_hbm.at[pl.ds(off, K)], sidx)
            pltpu.sync_copy(dst_hbm.at[pl.ds(off, K)], didx)
            pltpu.async_copy(g_hbm.at[sidx], rows, sem).wait()
            pltpu.sync_copy(rows, accum.at[didx], add=True)
            return 0

        lax.fori_loop(0, NFULL, chunk, 0)

        toff = ebase + NFULL * K
        pltpu.sync_copy(src_hbm.at[pl.ds(toff, TAIL)], tsidx)
        pltpu.sync_copy(dst_hbm.at[pl.ds(toff, TAIL)], tdidx)
        pltpu.async_copy(g_hbm.at[tsidx], trows, sem).wait()
        pltpu.sync_copy(trows, accum.at[tdidx], add=True)

        plsc.subcore_barrier()

        pltpu.sync_copy(accum.at[pl.ds(zbase, ZROWS)],
                        out_hbm.at[c, pl.ds(zbase, ZROWS)])

        @pl.when(s == NS - 1)
        def _():
            pltpu.sync_copy(accum.at[pl.ds(16 * ZROWS, 16)],
                            out_hbm.at[c, pl.ds(16 * ZROWS, 16)])

    return k(g, src, dst)


def _degrees(dst):
    DW = 16

    @functools.partial(
        pl.kernel,
        out_type=jax.ShapeDtypeStruct((NC, N, DW), FP32),
        mesh=_mesh(),
        compiler_params=pltpu.CompilerParams(use_tc_tiling_on_sc=False),
        scratch_types=[
            pltpu.VMEM((K,), I32),
            pltpu.VMEM((TAIL,), I32),
            pltpu.VMEM((K, DW), FP32),
            pltpu.VMEM((8, DW), FP32),
            pltpu.VMEM_SHARED((N, DW), FP32),
            pltpu.SemaphoreType.DMA,
        ],
    )
    def k(dst_hbm, out_hbm, didx, tdidx, ones, zbuf, accum, sem):
        c = lax.axis_index("c")
        s = lax.axis_index("s")
        wid = s * NC + c

        one16 = jnp.ones((16,), FP32)
        zero16 = jnp.zeros((16,), FP32)
        for r in range(K):
            ones[r, pl.ds(0, 16)] = one16
        for r in range(8):
            zbuf[r, pl.ds(0, 16)] = zero16

        zbase = s * ZROWS

        def zloop(i, _):
            pltpu.sync_copy(zbuf, accum.at[pl.ds(zbase + i * 8, 8)])
            return 0

        lax.fori_loop(0, ZROWS // 8, zloop, 0)

        @pl.when(s == NS - 1)
        def _():
            pltpu.sync_copy(zbuf, accum.at[pl.ds(16 * ZROWS, 8)])
            pltpu.sync_copy(zbuf, accum.at[pl.ds(16 * ZROWS + 8, 8)])

        plsc.subcore_barrier()

        ebase = wid * EPW

        def chunk(i, _):
            off = ebase + i * K
            pltpu.sync_copy(dst_hbm.at[pl.ds(off, K)], didx)
            pltpu.sync_copy(ones, accum.at[didx], add=True)
            return 0

        lax.fori_loop(0, NFULL, chunk, 0)

        toff = ebase + NFULL * K
        pltpu.sync_copy(dst_hbm.at[pl.ds(toff, TAIL)], tdidx)
        pltpu.sync_copy(ones.at[pl.ds(0, TAIL)], accum.at[tdidx], add=True)

        plsc.subcore_barrier()

        pltpu.sync_copy(accum.at[pl.ds(zbase, ZROWS)],
                        out_hbm.at[c, pl.ds(zbase, ZROWS)])

        @pl.when(s == NS - 1)
        def _():
            pltpu.sync_copy(accum.at[pl.ds(16 * ZROWS, 16)],
                            out_hbm.at[c, pl.ds(16 * ZROWS, 16)])

    return k(dst)


def _dec_gather(ab, src, dst):
    AW = 8

    @functools.partial(
        pl.kernel,
        out_type=jax.ShapeDtypeStruct((2, E, AW), FP32),
        mesh=_mesh(),
        compiler_params=pltpu.CompilerParams(use_tc_tiling_on_sc=False),
        scratch_types=[
            pltpu.VMEM((K,), I32),
            pltpu.VMEM((K,), I32),
            pltpu.VMEM((K, AW), FP32),
            pltpu.VMEM((K, AW), FP32),
            pltpu.VMEM((TAIL,), I32),
            pltpu.VMEM((TAIL,), I32),
            pltpu.VMEM((TAIL, AW), FP32),
            pltpu.SemaphoreType.DMA,
            pltpu.SemaphoreType.DMA,
        ],
    )
    def k(ab_hbm, src_hbm, dst_hbm, out_hbm,
          sidx, didx, srows, drows, tsidx, tdidx, trows, sem, sem2):
        c = lax.axis_index("c")
        s = lax.axis_index("s")
        wid = s * NC + c
        ebase = wid * EPW

        def chunk(i, _):
            off = ebase + i * K
            pltpu.sync_copy(src_hbm.at[pl.ds(off, K)], sidx)
            pltpu.sync_copy(dst_hbm.at[pl.ds(off, K)], didx)
            cp1 = pltpu.async_copy(ab_hbm.at[sidx], srows, sem)
            cp2 = pltpu.async_copy(ab_hbm.at[didx], drows, sem2)
            cp1.wait()
            pltpu.sync_copy(srows, out_hbm.at[0, pl.ds(off, K)])
            cp2.wait()
            pltpu.sync_copy(drows, out_hbm.at[1, pl.ds(off, K)])
            return 0

        lax.fori_loop(0, NFULL, chunk, 0)

        toff = ebase + NFULL * K
        pltpu.sync_copy(src_hbm.at[pl.ds(toff, TAIL)], tsidx)
        pltpu.async_copy(ab_hbm.at[tsidx], trows, sem).wait()
        pltpu.sync_copy(trows, out_hbm.at[0, pl.ds(toff, TAIL)])
        pltpu.sync_copy(dst_hbm.at[pl.ds(toff, TAIL)], tdidx)
        pltpu.async_copy(ab_hbm.at[tdidx], trows, sem).wait()
        pltpu.sync_copy(trows, out_hbm.at[1, pl.ds(toff, TAIL)])

    return k(ab, src, dst)


_NB = 5
_BR = N // _NB

_HI = jax.lax.Precision.HIGHEST


def _dot(a, b):
    return jnp.dot(a, b, precision=_HI, preferred_element_type=FP32)


def _encode(nf, W_enc, b_enc, W0, b0):
    def body(nf_ref, we_ref, be_ref, w0_ref, b0_ref, out_ref):
        h = jnp.maximum(_dot(nf_ref[...], we_ref[...]) + be_ref[...], 0.0)
        out_ref[...] = _dot(h, w0_ref[...]) + b0_ref[...]

    return pl.pallas_call(
        body,
        grid=(_NB,),
        in_specs=[
            pl.BlockSpec((_BR, 9), lambda i: (i, 0)),
            pl.BlockSpec((9, H), lambda i: (0, 0)),
            pl.BlockSpec((1, H), lambda i: (0, 0)),
            pl.BlockSpec((H, H), lambda i: (0, 0)),
            pl.BlockSpec((1, H), lambda i: (0, 0)),
        ],
        out_specs=pl.BlockSpec((_BR, H), lambda i: (i, 0)),
        out_shape=jax.ShapeDtypeStruct((N, H), FP32),
    )(nf, W_enc, b_enc, W0, b0)


def _layer_mid(P, degp, W, b):
    width = W.shape[1]

    def body(p_ref, d_ref, w_ref, b_ref, out_ref):
        m = p_ref[0] + p_ref[1]
        deg = d_ref[0, :, 0:1] + d_ref[1, :, 0:1]
        inv = 1.0 / jnp.maximum(deg, 1.0)
        h = jnp.maximum(m * inv, 0.0)
        out_ref[...] = _dot(h, w_ref[...]) + b_ref[...]

    return pl.pallas_call(
        body,
        grid=(_NB,),
        in_specs=[
            pl.BlockSpec((2, _BR, H), lambda i: (0, i, 0)),
            pl.BlockSpec((2, _BR, 16), lambda i: (0, i, 0)),
            pl.BlockSpec((H, width), lambda i: (0, 0)),
            pl.BlockSpec((1, width), lambda i: (0, 0)),
        ],
        out_specs=pl.BlockSpec((_BR, width), lambda i: (i, 0)),
        out_shape=jax.ShapeDtypeStruct((N, width), FP32),
    )(P, degp, W, b)


def _decode(uv, ef, W3, b_dec):
    NEB = 160
    BE = E // NEB

    def body(uv_ref, ef_ref, w3_ref, b_ref, lg_ref, pr_ref):
        au = uv_ref[0, :, 0:2]
        bv = uv_ref[1, :, 2:4]
        logits = _dot(ef_ref[...], w3_ref[...]) + b_ref[...] + au + bv
        lg_ref[...] = logits
        mx = jnp.max(logits, axis=1, keepdims=True)
        ex = jnp.exp(logits - mx)
        pr_ref[...] = ex / jnp.sum(ex, axis=1, keepdims=True)

    return pl.pallas_call(
        body,
        grid=(NEB,),
        in_specs=[
            pl.BlockSpec((2, BE, 8), lambda i: (0, i, 0)),
            pl.BlockSpec((BE, 14), lambda i: (i, 0)),
            pl.BlockSpec((14, 2), lambda i: (0, 0)),
            pl.BlockSpec((1, 2), lambda i: (0, 0)),
        ],
        out_specs=[
            pl.BlockSpec((BE, 2), lambda i: (i, 0)),
            pl.BlockSpec((BE, 2), lambda i: (i, 0)),
        ],
        out_shape=[
            jax.ShapeDtypeStruct((E, 2), FP32),
            jax.ShapeDtypeStruct((E, 2), FP32),
        ],
    )(uv, ef, W3, b_dec)


def kernel(node_feats, edge_feats, edge_index, W_enc, b_enc, Ws, bs, W_dec, b_dec):
    src = edge_index[0]
    dst = edge_index[1]

    Wd_uv = jnp.zeros((H, 8), FP32)
    Wd_uv = Wd_uv.at[:, 0:2].set(W_dec[0:H])
    Wd_uv = Wd_uv.at[:, 2:4].set(W_dec[H:2 * H])
    W3 = W_dec[2 * H:]

    degp = _degrees(dst)
    g = _encode(node_feats, W_enc, b_enc.reshape(1, H), Ws[0], bs[0].reshape(1, H))
    for i in range(1, L):
        P = _seg_sum(g, src, dst)
        g = _layer_mid(P, degp, Ws[i], bs[i].reshape(1, H))
    P = _seg_sum(g, src, dst)
    ab = _layer_mid(P, degp, Wd_uv, jnp.zeros((1, 8), FP32))

    uv = _dec_gather(ab, src, dst)
    logits, prob = _decode(uv, edge_feats, W3, b_dec.reshape(1, 2))
    return (logits, prob)


# speedup vs baseline: 6.4620x; 1.5137x over previous
import functools

import jax
import jax.numpy as jnp
from jax import lax
from jax.experimental import pallas as pl
from jax.experimental.pallas import tpu as pltpu
from jax.experimental.pallas import tpu_sc as plsc

N = 10000
E = 320000
H = 128
L = 10
NC = 2
NS = 16
NW = NC * NS
EPW = E // NW
K = 128
NFULL = EPW // K
TAIL = EPW - NFULL * K
ZROWS = 624

_mesh = lambda: plsc.VectorSubcoreMesh(core_axis_name="c", subcore_axis_name="s")

FP32 = jnp.float32
I32 = jnp.int32


KC = 80
NROWS = E // KC
RPW = NROWS // NW
SLOTS = 4
NBODY = 15


def _seg_sum(g, src2d, dst2d):
    @functools.partial(
        pl.kernel,
        out_type=jax.ShapeDtypeStruct((NC, N, H), FP32),
        mesh=_mesh(),
        compiler_params=pltpu.CompilerParams(use_tc_tiling_on_sc=False),
        scratch_types=[
            pltpu.VMEM((SLOTS, KC), I32),
            pltpu.VMEM((SLOTS, KC), I32),
            pltpu.VMEM((SLOTS, KC), I32),
            pltpu.VMEM((SLOTS, KC), I32),
            pltpu.VMEM((SLOTS, KC, H), FP32),
            pltpu.VMEM_SHARED((N, H), FP32),
            [pltpu.SemaphoreType.DMA] * 2,
            [pltpu.SemaphoreType.DMA] * 2,
            [pltpu.SemaphoreType.DMA] * SLOTS,
            [pltpu.SemaphoreType.DMA] * SLOTS,
        ],
    )
    def k(g_hbm, src_hbm, dst_hbm, out_hbm,
          sidxA, didxA, sidxB, didxB, rows, accum,
          isemA, isemB, gsems, ssems):
        c = lax.axis_index("c")
        s = lax.axis_index("s")
        wid = s * NC + c

        zero16 = jnp.zeros((16,), FP32)
        for r in range(KC):
            for q in range(H // 16):
                rows[0, r, pl.ds(q * 16, 16)] = zero16

        zbase = s * ZROWS

        def zloop(i, _):
            pltpu.sync_copy(rows.at[0], accum.at[pl.ds(zbase + i * KC, KC)])
            return 0

        lax.fori_loop(0, 7, zloop, 0)
        pltpu.sync_copy(rows.at[0, pl.ds(0, 64)],
                        accum.at[pl.ds(zbase + 560, 64)])

        @pl.when(s == NS - 1)
        def _():
            pltpu.sync_copy(rows.at[0, pl.ds(0, 16)],
                            accum.at[pl.ds(16 * ZROWS, 16)])

        plsc.subcore_barrier()

        rbase = wid * RPW

        def drain_scatter(j):
            pltpu.make_async_copy(g_hbm.at[pl.ds(0, KC)], rows.at[j],
                                  ssems[j]).wait()

        def fire_idx(roff, sidx, didx, isem):
            hs = pltpu.async_copy(src_hbm.at[pl.ds(roff, SLOTS)], sidx, isem[0])
            hd = pltpu.async_copy(dst_hbm.at[pl.ds(roff, SLOTS)], didx, isem[1])
            return hs, hd

        def do_half(hs, hd, sidx, didx, first):
            hs.wait()
            hg = []
            for j in range(SLOTS):
                if first is not None:
                    @pl.when(first)
                    def _():
                        drain_scatter(j)
                else:
                    drain_scatter(j)
                hg.append(pltpu.async_copy(g_hbm.at[sidx.at[j]], rows.at[j],
                                           gsems[j]))
            hd.wait()
            for j in range(SLOTS):
                hg[j].wait()
                pltpu.async_copy(rows.at[j], accum.at[didx.at[j]], ssems[j],
                                 add=True)

        def body(u, _):
            roff = rbase + u * (2 * SLOTS)
            hsA, hdA = fire_idx(roff, sidxA, didxA, isemA)
            do_half(hsA, hdA, sidxA, didxA, u > 0)
            hsB, hdB = fire_idx(roff + SLOTS, sidxB, didxB, isemB)
            do_half(hsB, hdB, sidxB, didxB, None)
            return 0

        lax.fori_loop(0, NBODY, body, 0)

        eoff = rbase + NBODY * 2 * SLOTS
        hsA, hdA = fire_idx(eoff, sidxA, didxA, isemA)
        do_half(hsA, hdA, sidxA, didxA, None)
        drain_scatter(0)
        pltpu.sync_copy(src_hbm.at[eoff + SLOTS], sidxB.at[0])
        pltpu.sync_copy(dst_hbm.at[eoff + SLOTS], didxB.at[0])
        pltpu.async_copy(g_hbm.at[sidxB.at[0]], rows.at[0], gsems[0]).wait()
        pltpu.sync_copy(rows.at[0], accum.at[didxB.at[0]], add=True)
        for j in range(1, SLOTS):
            drain_scatter(j)

        plsc.subcore_barrier()

        pltpu.sync_copy(accum.at[pl.ds(zbase, ZROWS)],
                        out_hbm.at[c, pl.ds(zbase, ZROWS)])

        @pl.when(s == NS - 1)
        def _():
            pltpu.sync_copy(accum.at[pl.ds(16 * ZROWS, 16)],
                            out_hbm.at[c, pl.ds(16 * ZROWS, 16)])

    return k(g, src2d, dst2d)


def _degrees(dst):
    DW = 16

    @functools.partial(
        pl.kernel,
        out_type=jax.ShapeDtypeStruct((NC, N, DW), FP32),
        mesh=_mesh(),
        compiler_params=pltpu.CompilerParams(use_tc_tiling_on_sc=False),
        scratch_types=[
            pltpu.VMEM((K,), I32),
            pltpu.VMEM((TAIL,), I32),
            pltpu.VMEM((K, DW), FP32),
            pltpu.VMEM((8, DW), FP32),
            pltpu.VMEM_SHARED((N, DW), FP32),
            pltpu.SemaphoreType.DMA,
        ],
    )
    def k(dst_hbm, out_hbm, didx, tdidx, ones, zbuf, accum, sem):
        c = lax.axis_index("c")
        s = lax.axis_index("s")
        wid = s * NC + c

        one16 = jnp.ones((16,), FP32)
        zero16 = jnp.zeros((16,), FP32)
        for r in range(K):
            ones[r, pl.ds(0, 16)] = one16
        for r in range(8):
            zbuf[r, pl.ds(0, 16)] = zero16

        zbase = s * ZROWS

        def zloop(i, _):
            pltpu.sync_copy(zbuf, accum.at[pl.ds(zbase + i * 8, 8)])
            return 0

        lax.fori_loop(0, ZROWS // 8, zloop, 0)

        @pl.when(s == NS - 1)
        def _():
            pltpu.sync_copy(zbuf, accum.at[pl.ds(16 * ZROWS, 8)])
            pltpu.sync_copy(zbuf, accum.at[pl.ds(16 * ZROWS + 8, 8)])

        plsc.subcore_barrier()

        ebase = wid * EPW

        def chunk(i, _):
            off = ebase + i * K
            pltpu.sync_copy(dst_hbm.at[pl.ds(off, K)], didx)
            pltpu.sync_copy(ones, accum.at[didx], add=True)
            return 0

        lax.fori_loop(0, NFULL, chunk, 0)

        toff = ebase + NFULL * K
        pltpu.sync_copy(dst_hbm.at[pl.ds(toff, TAIL)], tdidx)
        pltpu.sync_copy(ones.at[pl.ds(0, TAIL)], accum.at[tdidx], add=True)

        plsc.subcore_barrier()

        pltpu.sync_copy(accum.at[pl.ds(zbase, ZROWS)],
                        out_hbm.at[c, pl.ds(zbase, ZROWS)])

        @pl.when(s == NS - 1)
        def _():
            pltpu.sync_copy(accum.at[pl.ds(16 * ZROWS, 16)],
                            out_hbm.at[c, pl.ds(16 * ZROWS, 16)])

    return k(dst)


def _dec_gather(ab, src, dst):
    AW = 8

    @functools.partial(
        pl.kernel,
        out_type=jax.ShapeDtypeStruct((2, E, AW), FP32),
        mesh=_mesh(),
        compiler_params=pltpu.CompilerParams(use_tc_tiling_on_sc=False),
        scratch_types=[
            pltpu.VMEM((K,), I32),
            pltpu.VMEM((K,), I32),
            pltpu.VMEM((K, AW), FP32),
            pltpu.VMEM((K, AW), FP32),
            pltpu.VMEM((TAIL,), I32),
            pltpu.VMEM((TAIL,), I32),
            pltpu.VMEM((TAIL, AW), FP32),
            pltpu.SemaphoreType.DMA,
            pltpu.SemaphoreType.DMA,
        ],
    )
    def k(ab_hbm, src_hbm, dst_hbm, out_hbm,
          sidx, didx, srows, drows, tsidx, tdidx, trows, sem, sem2):
        c = lax.axis_index("c")
        s = lax.axis_index("s")
        wid = s * NC + c
        ebase = wid * EPW

        def chunk(i, _):
            off = ebase + i * K
            pltpu.sync_copy(src_hbm.at[pl.ds(off, K)], sidx)
            pltpu.sync_copy(dst_hbm.at[pl.ds(off, K)], didx)
            cp1 = pltpu.async_copy(ab_hbm.at[sidx], srows, sem)
            cp2 = pltpu.async_copy(ab_hbm.at[didx], drows, sem2)
            cp1.wait()
            pltpu.sync_copy(srows, out_hbm.at[0, pl.ds(off, K)])
            cp2.wait()
            pltpu.sync_copy(drows, out_hbm.at[1, pl.ds(off, K)])
            return 0

        lax.fori_loop(0, NFULL, chunk, 0)

        toff = ebase + NFULL * K
        pltpu.sync_copy(src_hbm.at[pl.ds(toff, TAIL)], tsidx)
        pltpu.async_copy(ab_hbm.at[tsidx], trows, sem).wait()
        pltpu.sync_copy(trows, out_hbm.at[0, pl.ds(toff, TAIL)])
        pltpu.sync_copy(dst_hbm.at[pl.ds(toff, TAIL)], tdidx)
        pltpu.async_copy(ab_hbm.at[tdidx], trows, sem).wait()
        pltpu.sync_copy(trows, out_hbm.at[1, pl.ds(toff, TAIL)])

    return k(ab, src, dst)


_NB = 5
_BR = N // _NB

_HI = jax.lax.Precision.HIGHEST


def _dot(a, b):
    return jnp.dot(a, b, precision=_HI, preferred_element_type=FP32)


def _encode(nf, W_enc, b_enc, W0, b0):
    def body(nf_ref, we_ref, be_ref, w0_ref, b0_ref, out_ref):
        h = jnp.maximum(_dot(nf_ref[...], we_ref[...]) + be_ref[...], 0.0)
        out_ref[...] = _dot(h, w0_ref[...]) + b0_ref[...]

    return pl.pallas_call(
        body,
        grid=(_NB,),
        in_specs=[
            pl.BlockSpec((_BR, 9), lambda i: (i, 0)),
            pl.BlockSpec((9, H), lambda i: (0, 0)),
            pl.BlockSpec((1, H), lambda i: (0, 0)),
            pl.BlockSpec((H, H), lambda i: (0, 0)),
            pl.BlockSpec((1, H), lambda i: (0, 0)),
        ],
        out_specs=pl.BlockSpec((_BR, H), lambda i: (i, 0)),
        out_shape=jax.ShapeDtypeStruct((N, H), FP32),
    )(nf, W_enc, b_enc, W0, b0)


def _layer_mid(P, degp, W, b):
    width = W.shape[1]

    def body(p_ref, d_ref, w_ref, b_ref, out_ref):
        m = p_ref[0] + p_ref[1]
        deg = d_ref[0, :, 0:1] + d_ref[1, :, 0:1]
        inv = 1.0 / jnp.maximum(deg, 1.0)
        h = jnp.maximum(m * inv, 0.0)
        out_ref[...] = _dot(h, w_ref[...]) + b_ref[...]

    return pl.pallas_call(
        body,
        grid=(_NB,),
        in_specs=[
            pl.BlockSpec((2, _BR, H), lambda i: (0, i, 0)),
            pl.BlockSpec((2, _BR, 16), lambda i: (0, i, 0)),
            pl.BlockSpec((H, width), lambda i: (0, 0)),
            pl.BlockSpec((1, width), lambda i: (0, 0)),
        ],
        out_specs=pl.BlockSpec((_BR, width), lambda i: (i, 0)),
        out_shape=jax.ShapeDtypeStruct((N, width), FP32),
    )(P, degp, W, b)


def _decode(uv, ef, W3, b_dec):
    NEB = 160
    BE = E // NEB

    def body(uv_ref, ef_ref, w3_ref, b_ref, lg_ref, pr_ref):
        au = uv_ref[0, :, 0:2]
        bv = uv_ref[1, :, 2:4]
        logits = _dot(ef_ref[...], w3_ref[...]) + b_ref[...] + au + bv
        lg_ref[...] = logits
        mx = jnp.max(logits, axis=1, keepdims=True)
        ex = jnp.exp(logits - mx)
        pr_ref[...] = ex / jnp.sum(ex, axis=1, keepdims=True)

    return pl.pallas_call(
        body,
        grid=(NEB,),
        in_specs=[
            pl.BlockSpec((2, BE, 8), lambda i: (0, i, 0)),
            pl.BlockSpec((BE, 14), lambda i: (i, 0)),
            pl.BlockSpec((14, 2), lambda i: (0, 0)),
            pl.BlockSpec((1, 2), lambda i: (0, 0)),
        ],
        out_specs=[
            pl.BlockSpec((BE, 2), lambda i: (i, 0)),
            pl.BlockSpec((BE, 2), lambda i: (i, 0)),
        ],
        out_shape=[
            jax.ShapeDtypeStruct((E, 2), FP32),
            jax.ShapeDtypeStruct((E, 2), FP32),
        ],
    )(uv, ef, W3, b_dec)


def kernel(node_feats, edge_feats, edge_index, W_enc, b_enc, Ws, bs, W_dec, b_dec):
    src = edge_index[0]
    dst = edge_index[1]

    Wd_uv = jnp.zeros((H, 8), FP32)
    Wd_uv = Wd_uv.at[:, 0:2].set(W_dec[0:H])
    Wd_uv = Wd_uv.at[:, 2:4].set(W_dec[H:2 * H])
    W3 = W_dec[2 * H:]

    src2d = src.reshape(NROWS, KC)
    dst2d = dst.reshape(NROWS, KC)

    degp = _degrees(dst)
    g = _encode(node_feats, W_enc, b_enc.reshape(1, H), Ws[0], bs[0].reshape(1, H))
    for i in range(1, L):
        P = _seg_sum(g, src2d, dst2d)
        g = _layer_mid(P, degp, Ws[i], bs[i].reshape(1, H))
    P = _seg_sum(g, src2d, dst2d)
    ab = _layer_mid(P, degp, Wd_uv, jnp.zeros((1, 8), FP32))

    uv = _dec_gather(ab, src, dst)
    logits, prob = _decode(uv, edge_feats, W3, b_dec.reshape(1, 2))
    return (logits, prob)


# speedup vs baseline: 6.7047x; 1.0376x over previous
import functools

import jax
import jax.numpy as jnp
from jax import lax
from jax.experimental import pallas as pl
from jax.experimental.pallas import tpu as pltpu
from jax.experimental.pallas import tpu_sc as plsc

N = 10000
E = 320000
H = 128
L = 10
NC = 2
NS = 16
NW = NC * NS
EPW = E // NW
K = 128
NFULL = EPW // K
TAIL = EPW - NFULL * K
ZROWS = 624

_mesh = lambda: plsc.VectorSubcoreMesh(core_axis_name="c", subcore_axis_name="s")

FP32 = jnp.float32
I32 = jnp.int32


HH = H // 2
NROWS = E // K
RPW = NROWS // NW
NEXTRA = NROWS - RPW * NW
SLOTS = 6
RPT = NROWS // NS
NTEXTRA = NROWS - RPT * NS
NBODY = 13


def _seg_sum(g2, src2_2d, dst2d):
    @functools.partial(
        pl.kernel,
        out_type=jax.ShapeDtypeStruct((NC, N, HH), FP32),
        mesh=_mesh(),
        compiler_params=pltpu.CompilerParams(use_tc_tiling_on_sc=False),
        scratch_types=[
            pltpu.VMEM((SLOTS, K), I32),
            pltpu.VMEM((SLOTS, K), I32),
            pltpu.VMEM((SLOTS, K), I32),
            pltpu.VMEM((SLOTS, K), I32),
            pltpu.VMEM((SLOTS, K, HH), FP32),
            pltpu.VMEM_SHARED((N, HH), FP32),
            [pltpu.SemaphoreType.DMA] * 2,
            [pltpu.SemaphoreType.DMA] * 2,
            [pltpu.SemaphoreType.DMA] * SLOTS,
            [pltpu.SemaphoreType.DMA] * SLOTS,
        ],
    )
    def k(g_hbm, src_hbm, dst_hbm, out_hbm,
          sidxA, didxA, sidxB, didxB, rows, accum,
          isemA, isemB, gsems, ssems):
        c = lax.axis_index("c")
        s = lax.axis_index("s")
        wid = s * NC + c

        zero16 = jnp.zeros((16,), FP32)
        for r in range(K):
            for q in range(HH // 16):
                rows[0, r, pl.ds(q * 16, 16)] = zero16

        zbase = s * ZROWS

        def zloop(i, _):
            pltpu.sync_copy(rows.at[0], accum.at[pl.ds(zbase + i * K, K)])
            return 0

        lax.fori_loop(0, 4, zloop, 0)
        pltpu.sync_copy(rows.at[0, pl.ds(0, 112)],
                        accum.at[pl.ds(zbase + 512, 112)])

        @pl.when(s == NS - 1)
        def _():
            pltpu.sync_copy(rows.at[0, pl.ds(0, 16)],
                            accum.at[pl.ds(16 * ZROWS, 16)])

        plsc.subcore_barrier()

        rbase = s * RPT

        def adjust_src(sidx, j):
            for q in range(K // 16):
                sidx[j, pl.ds(q * 16, 16)] = sidx[j, pl.ds(q * 16, 16)] + c

        def drain_scatter(j):
            pltpu.make_async_copy(g_hbm.at[pl.ds(0, K)], rows.at[j],
                                  ssems[j]).wait()

        def fire_idx(roff, sidx, didx, isem):
            hs = pltpu.async_copy(src_hbm.at[pl.ds(roff, SLOTS)], sidx, isem[0])
            hd = pltpu.async_copy(dst_hbm.at[pl.ds(roff, SLOTS)], didx, isem[1])
            return hs, hd

        def do_half(hs, hd, sidx, didx, first):
            hs.wait()
            hg = []
            for j in range(SLOTS):
                adjust_src(sidx, j)
                if first is not None:
                    @pl.when(first)
                    def _():
                        drain_scatter(j)
                else:
                    drain_scatter(j)
                hg.append(pltpu.async_copy(g_hbm.at[sidx.at[j]], rows.at[j],
                                           gsems[j]))
            hd.wait()
            for j in range(SLOTS):
                hg[j].wait()
                pltpu.async_copy(rows.at[j], accum.at[didx.at[j]], ssems[j],
                                 add=True)

        def body(u, _):
            roff = rbase + u * (2 * SLOTS)
            hsA, hdA = fire_idx(roff, sidxA, didxA, isemA)
            do_half(hsA, hdA, sidxA, didxA, u > 0)
            hsB, hdB = fire_idx(roff + SLOTS, sidxB, didxB, isemB)
            do_half(hsB, hdB, sidxB, didxB, None)
            return 0

        lax.fori_loop(0, NBODY, body, 0)

        @pl.when(s < NTEXTRA)
        def _():
            xrow = NS * RPT + s
            drain_scatter(0)
            pltpu.sync_copy(src_hbm.at[xrow], sidxB.at[0])
            pltpu.sync_copy(dst_hbm.at[xrow], didxB.at[0])
            adjust_src(sidxB, 0)
            pltpu.async_copy(g_hbm.at[sidxB.at[0]], rows.at[0],
                             gsems[0]).wait()
            pltpu.sync_copy(rows.at[0], accum.at[didxB.at[0]], add=True)

        @pl.when(s >= NTEXTRA)
        def _():
            drain_scatter(0)
        for j in range(1, SLOTS):
            drain_scatter(j)

        plsc.subcore_barrier()

        pltpu.sync_copy(accum.at[pl.ds(zbase, ZROWS)],
                        out_hbm.at[c, pl.ds(zbase, ZROWS)])

        @pl.when(s == NS - 1)
        def _():
            pltpu.sync_copy(accum.at[pl.ds(16 * ZROWS, 16)],
                            out_hbm.at[c, pl.ds(16 * ZROWS, 16)])

    return k(g2, src2_2d, dst2d)


def _degrees(dst):
    DW = 16

    @functools.partial(
        pl.kernel,
        out_type=jax.ShapeDtypeStruct((NC, N, DW), FP32),
        mesh=_mesh(),
        compiler_params=pltpu.CompilerParams(use_tc_tiling_on_sc=False),
        scratch_types=[
            pltpu.VMEM((K,), I32),
            pltpu.VMEM((TAIL,), I32),
            pltpu.VMEM((K, DW), FP32),
            pltpu.VMEM((8, DW), FP32),
            pltpu.VMEM_SHARED((N, DW), FP32),
            pltpu.SemaphoreType.DMA,
        ],
    )
    def k(dst_hbm, out_hbm, didx, tdidx, ones, zbuf, accum, sem):
        c = lax.axis_index("c")
        s = lax.axis_index("s")
        wid = s * NC + c

        one16 = jnp.ones((16,), FP32)
        zero16 = jnp.zeros((16,), FP32)
        for r in range(K):
            ones[r, pl.ds(0, 16)] = one16
        for r in range(8):
            zbuf[r, pl.ds(0, 16)] = zero16

        zbase = s * ZROWS

        def zloop(i, _):
            pltpu.sync_copy(zbuf, accum.at[pl.ds(zbase + i * 8, 8)])
            return 0

        lax.fori_loop(0, ZROWS // 8, zloop, 0)

        @pl.when(s == NS - 1)
        def _():
            pltpu.sync_copy(zbuf, accum.at[pl.ds(16 * ZROWS, 8)])
            pltpu.sync_copy(zbuf, accum.at[pl.ds(16 * ZROWS + 8, 8)])

        plsc.subcore_barrier()

        ebase = wid * EPW

        def chunk(i, _):
            off = ebase + i * K
            pltpu.sync_copy(dst_hbm.at[pl.ds(off, K)], didx)
            pltpu.sync_copy(ones, accum.at[didx], add=True)
            return 0

        lax.fori_loop(0, NFULL, chunk, 0)

        toff = ebase + NFULL * K
        pltpu.sync_copy(dst_hbm.at[pl.ds(toff, TAIL)], tdidx)
        pltpu.sync_copy(ones.at[pl.ds(0, TAIL)], accum.at[tdidx], add=True)

        plsc.subcore_barrier()

        pltpu.sync_copy(accum.at[pl.ds(zbase, ZROWS)],
                        out_hbm.at[c, pl.ds(zbase, ZROWS)])

        @pl.when(s == NS - 1)
        def _():
            pltpu.sync_copy(accum.at[pl.ds(16 * ZROWS, 16)],
                            out_hbm.at[c, pl.ds(16 * ZROWS, 16)])

    return k(dst)


def _dec_gather(ab, src2d_, dst2d_):
    AW = 8
    DRPW = 2 * RPW
    DBODY = 13

    @functools.partial(
        pl.kernel,
        out_type=jax.ShapeDtypeStruct((2, E, AW), FP32),
        mesh=_mesh(),
        compiler_params=pltpu.CompilerParams(use_tc_tiling_on_sc=False),
        scratch_types=[
            pltpu.VMEM((SLOTS, K), I32),
            pltpu.VMEM((SLOTS, K), I32),
            pltpu.VMEM((SLOTS, K, AW), FP32),
            [pltpu.SemaphoreType.DMA] * 2,
            [pltpu.SemaphoreType.DMA] * SLOTS,
            [pltpu.SemaphoreType.DMA] * SLOTS,
        ],
    )
    def k(ab_hbm, src_hbm, dst_hbm, out_hbm,
          idxA, idxB, rows, isems, gsems, wsems):
        c = lax.axis_index("c")
        s = lax.axis_index("s")
        wid = s * NC + c
        half = wid % 2
        pos = wid // 2
        rbase = pos * DRPW

        def drain_write(j):
            pltpu.make_async_copy(ab_hbm.at[pl.ds(0, K)], rows.at[j],
                                  wsems[j]).wait()

        def fire_idx(roff, idx, isem, ihbm):
            return pltpu.async_copy(ihbm.at[pl.ds(roff, SLOTS)], idx, isem)

        def do_half(h, idx, roff, first, ihbm):
            h.wait()
            hg = []
            for j in range(SLOTS):
                if first is not None:
                    @pl.when(first)
                    def _():
                        drain_write(j)
                else:
                    drain_write(j)
                hg.append(pltpu.async_copy(ab_hbm.at[idx.at[j]], rows.at[j],
                                           gsems[j]))
            for j in range(SLOTS):
                hg[j].wait()
                pltpu.async_copy(rows.at[j],
                                 out_hbm.at[half, pl.ds((roff + j) * K, K)],
                                 wsems[j])

        def run(ihbm):
            def body(u, _):
                roff = rbase + u * (2 * SLOTS)
                hA = fire_idx(roff, idxA, isems[0], ihbm)
                do_half(hA, idxA, roff, u > 0, ihbm)
                hB = fire_idx(roff + SLOTS, idxB, isems[1], ihbm)
                do_half(hB, idxB, roff + SLOTS, None, ihbm)
                return 0

            lax.fori_loop(0, DBODY, body, 0)
            @pl.when(pos < 2)
            def _():
                for j in range(2):
                    xrow = NW * RPW + pos * 2 + j
                    drain_write(j)
                    pltpu.sync_copy(ihbm.at[xrow], idxA.at[j])
                    pltpu.async_copy(ab_hbm.at[idxA.at[j]], rows.at[j],
                                     gsems[j]).wait()
                    pltpu.sync_copy(rows.at[j],
                                    out_hbm.at[half, pl.ds(xrow * K, K)])

            @pl.when(pos >= 2)
            def _():
                for j in range(2):
                    drain_write(j)
            for j in range(2, SLOTS):
                drain_write(j)

        @pl.when(half == 0)
        def _():
            run(src_hbm)

        @pl.when(half == 1)
        def _():
            run(dst_hbm)

    return k(ab, src2d_, dst2d_)


_NB = 5
_BR = N // _NB

_HI = jax.lax.Precision.HIGHEST


def _dot(a, b):
    return jnp.dot(a, b, precision=_HI, preferred_element_type=FP32)


def _encode(nf, W_enc, b_enc, W0, b0):
    def body(nf_ref, we_ref, be_ref, w0_ref, b0_ref, out_ref):
        h = jnp.maximum(_dot(nf_ref[...], we_ref[...]) + be_ref[...], 0.0)
        out_ref[...] = _dot(h, w0_ref[...]) + b0_ref[...]

    return pl.pallas_call(
        body,
        grid=(_NB,),
        in_specs=[
            pl.BlockSpec((_BR, 9), lambda i: (i, 0)),
            pl.BlockSpec((9, H), lambda i: (0, 0)),
            pl.BlockSpec((1, H), lambda i: (0, 0)),
            pl.BlockSpec((H, H), lambda i: (0, 0)),
            pl.BlockSpec((1, H), lambda i: (0, 0)),
        ],
        out_specs=pl.BlockSpec((_BR, H), lambda i: (i, 0)),
        out_shape=jax.ShapeDtypeStruct((N, H), FP32),
    )(nf, W_enc, b_enc, W0, b0)


def _layer_mid(P, degp, W, b):
    width = W.shape[1]

    def body(p_ref, d_ref, w_ref, b_ref, out_ref):
        m = jnp.concatenate([p_ref[0], p_ref[1]], axis=1)
        deg = d_ref[0, :, 0:1] + d_ref[1, :, 0:1]
        inv = 1.0 / jnp.maximum(deg, 1.0)
        h = jnp.maximum(m * inv, 0.0)
        out_ref[...] = _dot(h, w_ref[...]) + b_ref[...]

    return pl.pallas_call(
        body,
        grid=(_NB,),
        in_specs=[
            pl.BlockSpec((2, _BR, HH), lambda i: (0, i, 0)),
            pl.BlockSpec((2, _BR, 16), lambda i: (0, i, 0)),
            pl.BlockSpec((H, width), lambda i: (0, 0)),
            pl.BlockSpec((1, width), lambda i: (0, 0)),
        ],
        out_specs=pl.BlockSpec((_BR, width), lambda i: (i, 0)),
        out_shape=jax.ShapeDtypeStruct((N, width), FP32),
    )(P, degp, W, b)


def _decode(uv, ef, W3, b_dec):
    NEB = 160
    BE = E // NEB

    def body(uv_ref, ef_ref, w3_ref, b_ref, lg_ref, pr_ref):
        au = uv_ref[0, :, 0:2]
        bv = uv_ref[1, :, 2:4]
        logits = _dot(ef_ref[...], w3_ref[...]) + b_ref[...] + au + bv
        lg_ref[...] = logits
        mx = jnp.max(logits, axis=1, keepdims=True)
        ex = jnp.exp(logits - mx)
        pr_ref[...] = ex / jnp.sum(ex, axis=1, keepdims=True)

    return pl.pallas_call(
        body,
        grid=(NEB,),
        in_specs=[
            pl.BlockSpec((2, BE, 8), lambda i: (0, i, 0)),
            pl.BlockSpec((BE, 14), lambda i: (i, 0)),
            pl.BlockSpec((14, 2), lambda i: (0, 0)),
            pl.BlockSpec((1, 2), lambda i: (0, 0)),
        ],
        out_specs=[
            pl.BlockSpec((BE, 2), lambda i: (i, 0)),
            pl.BlockSpec((BE, 2), lambda i: (i, 0)),
        ],
        out_shape=[
            jax.ShapeDtypeStruct((E, 2), FP32),
            jax.ShapeDtypeStruct((E, 2), FP32),
        ],
    )(uv, ef, W3, b_dec)


def kernel(node_feats, edge_feats, edge_index, W_enc, b_enc, Ws, bs, W_dec, b_dec):
    src = edge_index[0]
    dst = edge_index[1]

    Wd_uv = jnp.zeros((H, 8), FP32)
    Wd_uv = Wd_uv.at[:, 0:2].set(W_dec[0:H])
    Wd_uv = Wd_uv.at[:, 2:4].set(W_dec[H:2 * H])
    W3 = W_dec[2 * H:]

    src2d = src.reshape(NROWS, K)
    dst2d = dst.reshape(NROWS, K)
    src2_2d = (2 * src).reshape(NROWS, K)

    degp = _degrees(dst)
    g = _encode(node_feats, W_enc, b_enc.reshape(1, H), Ws[0], bs[0].reshape(1, H))
    for i in range(1, L):
        P = _seg_sum(g.reshape(2 * N, HH), src2_2d, dst2d)
        g = _layer_mid(P, degp, Ws[i], bs[i].reshape(1, H))
    P = _seg_sum(g.reshape(2 * N, HH), src2_2d, dst2d)
    ab = _layer_mid(P, degp, Wd_uv, jnp.zeros((1, 8), FP32))

    uv = _dec_gather(ab, src2d, dst2d)
    logits, prob = _decode(uv, edge_feats, W3, b_dec.reshape(1, 2))
    return (logits, prob)
